# baseline (speedup 1.0000x reference)
import functools

import jax
import jax.numpy as jnp
from jax import lax
from jax.experimental import pallas as pl
from jax.experimental.pallas import tpu as pltpu
from jax.experimental.pallas import tpu_sc as plsc

N = 10000
NPAD = 10112
E = 320000
D = 128
NC = 2
NS = 16
NW = NC * NS
EP = E // NW
K = 80
NCHUNK = EP // K
RP = NPAD // NS
RCS = (80, 80, 80, 80, 80, 80, 80, 72)

_mesh = plsc.VectorSubcoreMesh(core_axis_name="c", subcore_axis_name="s")
_sc_params = pltpu.CompilerParams(needs_layout_passes=False)


@functools.partial(
    pl.kernel,
    out_type=jax.ShapeDtypeStruct((NW, NPAD), jnp.float32),
    mesh=_mesh,
    scratch_types=[
        pltpu.VMEM((EP,), jnp.int32),
        pltpu.VMEM((NPAD,), jnp.float32),
    ],
    compiler_params=_sc_params,
)
def _count(dst_hbm, cnt_hbm, dst_v, cnt_v):
    c = lax.axis_index("c")
    s = lax.axis_index("s")
    wid = c * NS + s

    def _zcnt(i, _):
        cnt_v[pl.ds(i * 16, 16)] = jnp.zeros((16,), jnp.float32)
        return 0
    lax.fori_loop(0, NPAD // 16, _zcnt, 0)

    pltpu.sync_copy(dst_hbm.at[wid], dst_v)

    ones16 = jnp.ones((16,), jnp.float32)

    def _body(i, _):
        di = dst_v[pl.ds(i * 16, 16)]
        plsc.addupdate_scatter(cnt_v, [di], ones16)
        return 0
    lax.fori_loop(0, EP // 16, _body, 0)

    pltpu.sync_copy(cnt_v, cnt_hbm.at[wid])


@functools.partial(
    pl.kernel,
    out_type=jax.ShapeDtypeStruct((NC, NPAD, D), jnp.float32),
    mesh=_mesh,
    scratch_types=[
        pltpu.VMEM((NCHUNK, K), jnp.int32),
        pltpu.VMEM((K,), jnp.int32),
        pltpu.VMEM((K,), jnp.int32),
        pltpu.VMEM((K, D), jnp.float32),
        pltpu.VMEM((K,), jnp.int32),
        pltpu.VMEM((K,), jnp.int32),
        pltpu.VMEM((K, D), jnp.float32),
        pltpu.VMEM_SHARED((NPAD, D), jnp.float32),
        pltpu.SemaphoreType.DMA,
        pltpu.SemaphoreType.DMA,
        pltpu.SemaphoreType.DMA,
        pltpu.SemaphoreType.DMA,
    ],
    compiler_params=_sc_params,
)
def _segsum(tab_hbm, edges_hbm, out_hbm,
            edges_v, src_a, dst_a, rows_a, src_b, dst_b, rows_b,
            acc_shared, sem_a, sem_b, ssem_a, ssem_b):
    c = lax.axis_index("c")
    s = lax.axis_index("s")
    wid = c * NS + s

    def _zrow(i, _):
        for j in range(D // 16):
            rows_a[i, pl.ds(j * 16, 16)] = jnp.zeros((16,), jnp.float32)
        return 0
    lax.fori_loop(0, K, _zrow, 0)
    off = 0
    for rc in RCS:
        pltpu.sync_copy(rows_a.at[pl.ds(0, rc), :],
                        acc_shared.at[pl.ds(s * RP + off, rc), :])
        off += rc

    pltpu.sync_copy(edges_hbm.at[wid], edges_v)

    plsc.subcore_barrier()

    def _prefetch(j, src_c, dst_c, rows_c, sem_c):
        for m in range(K // 16):
            p = edges_v[j, pl.ds(m * 16, 16)]
            src_c[pl.ds(m * 16, 16)] = jnp.bitwise_and(p, 16383)
            dst_c[pl.ds(m * 16, 16)] = lax.shift_right_logical(p, 14)
        pltpu.async_copy(tab_hbm.at[src_c], rows_c, sem_c)

    def _wait_gather(src_c, rows_c, sem_c):
        pltpu.make_async_copy(tab_hbm.at[src_c], rows_c, sem_c).wait()

    def _scatter(dst_c, rows_c, ssem_c):
        pltpu.async_copy(rows_c, acc_shared.at[dst_c], ssem_c, add=True)

    def _wait_scatter(dst_c, rows_c, ssem_c):
        pltpu.make_async_copy(rows_c, acc_shared.at[dst_c], ssem_c).wait()

    _prefetch(0, src_a, dst_a, rows_a, sem_a)

    def _pair(i, _):
        ja = 2 * i

        @pl.when(ja >= 1)
        def _():
            _wait_scatter(dst_b, rows_b, ssem_b)
        _prefetch(ja + 1, src_b, dst_b, rows_b, sem_b)
        _wait_gather(src_a, rows_a, sem_a)
        _scatter(dst_a, rows_a, ssem_a)
        _wait_gather(src_b, rows_b, sem_b)
        _scatter(dst_b, rows_b, ssem_b)
        _wait_scatter(dst_a, rows_a, ssem_a)

        @pl.when(ja + 2 < NCHUNK)
        def _():
            _prefetch(ja + 2, src_a, dst_a, rows_a, sem_a)
        return 0
    lax.fori_loop(0, NCHUNK // 2, _pair, 0)

    _wait_gather(src_a, rows_a, sem_a)
    _scatter(dst_a, rows_a, ssem_a)
    _wait_scatter(dst_b, rows_b, ssem_b)
    _wait_scatter(dst_a, rows_a, ssem_a)

    plsc.subcore_barrier()

    off = 0
    bufs = (rows_a, rows_b)
    for r, rc in enumerate(RCS):
        row0 = s * RP + off
        buf = bufs[r % 2]
        pltpu.sync_copy(acc_shared.at[pl.ds(row0, rc), :],
                        buf.at[pl.ds(0, rc), :])
        pltpu.sync_copy(buf.at[pl.ds(0, rc), :],
                        out_hbm.at[c, pl.ds(row0, rc), :])
        off += rc


@functools.partial(
    pl.kernel,
    out_type=jax.ShapeDtypeStruct((E,), jnp.float32),
    mesh=_mesh,
    scratch_types=[
        pltpu.VMEM((N,), jnp.float32),
        pltpu.VMEM((N,), jnp.float32),
        pltpu.VMEM((EP,), jnp.int32),
        pltpu.VMEM((EP,), jnp.int32),
        pltpu.VMEM((EP,), jnp.float32),
        pltpu.VMEM((2 * D,), jnp.float32),
        pltpu.VMEM((D,), jnp.float32),
        pltpu.VMEM((16,), jnp.float32),
    ],
    compiler_params=_sc_params,
)
def _head(s_hbm, t_hbm, src_hbm, dst_hbm, we_hbm, b2_hbm, be_hbm, out_hbm,
          s_v, t_v, src_v, dst_v, y_v, we_v, b2_v, be_v):
    c = lax.axis_index("c")
    s = lax.axis_index("s")
    wid = c * NS + s
    base = wid * EP

    pltpu.sync_copy(s_hbm, s_v)
    pltpu.sync_copy(t_hbm, t_v)
    pltpu.sync_copy(src_hbm.at[pl.ds(base, EP)], src_v)
    pltpu.sync_copy(dst_hbm.at[pl.ds(base, EP)], dst_v)
    pltpu.sync_copy(we_hbm, we_v)
    pltpu.sync_copy(b2_hbm, b2_v)
    pltpu.sync_copy(be_hbm, be_v)

    acc = be_v[...]
    for m in range(D // 16):
        b2c = b2_v[pl.ds(m * 16, 16)]
        acc = acc + b2c * (we_v[pl.ds(m * 16, 16)]
                           + we_v[pl.ds(D + m * 16, 16)])
    cconst = jnp.sum(acc, axis=0)

    def _body(i, _):
        si = src_v[pl.ds(i * 16, 16)]
        di = dst_v[pl.ds(i * 16, 16)]
        sv = plsc.load_gather(s_v, [si])
        tv = plsc.load_gather(t_v, [di])
        logit = sv + tv + cconst
        y_v[pl.ds(i * 16, 16)] = 1.0 / (1.0 + jnp.exp(-logit))
        return 0
    lax.fori_loop(0, EP // 16, _body, 0)

    pltpu.sync_copy(y_v, out_hbm.at[pl.ds(base, EP)])


def _tc1_body(p_ref, cnt_ref, x_ref, wl_ref, b_ref, wr_ref, out_ref):
    cnt = jnp.sum(cnt_ref[...], axis=0)
    inv = (1.0 / jnp.maximum(cnt, 1.0))[:, None]
    mean = (p_ref[0] + p_ref[1]) * inv
    h = lax.dot_general(mean, wl_ref[...], (((1,), (1,)), ((), ())),
                        preferred_element_type=jnp.float32)
    h = h + b_ref[...] + lax.dot_general(x_ref[...], wr_ref[...],
                                         (((1,), (1,)), ((), ())),
                                         preferred_element_type=jnp.float32)
    out_ref[...] = jnp.maximum(h, 0.0)


def _tc2_body(q_ref, cnt_ref, h_ref, wl_ref, wr_ref, we_ref, out_ref):
    cnt = jnp.sum(cnt_ref[...], axis=0)
    inv = (1.0 / jnp.maximum(cnt, 1.0))[:, None]
    mean2 = (q_ref[0] + q_ref[1]) * inv
    h = h_ref[...]
    wel = we_ref[:, :D]
    wer = we_ref[:, D:]
    cdims = (((1,), (0,)), ((), ()))

    a_s = lax.dot_general(wel, wl_ref[...], cdims,
                          preferred_element_type=jnp.float32)
    b_s = lax.dot_general(wel, wr_ref[...], cdims,
                          preferred_element_type=jnp.float32)
    a_t = lax.dot_general(wer, wl_ref[...], cdims,
                          preferred_element_type=jnp.float32)
    b_t = lax.dot_general(wer, wr_ref[...], cdims,
                          preferred_element_type=jnp.float32)
    rdims = (((1,), (1,)), ((), ()))

    scol = (lax.dot_general(mean2, a_s, rdims, preferred_element_type=jnp.float32)
            + lax.dot_general(h, b_s, rdims, preferred_element_type=jnp.float32))
    tcol = (lax.dot_general(mean2, a_t, rdims, preferred_element_type=jnp.float32)
            + lax.dot_general(h, b_t, rdims, preferred_element_type=jnp.float32))
    lane = lax.broadcasted_iota(jnp.int32, out_ref.shape, 1)
    out_ref[...] = jnp.where(lane == 0, scol, 0.0) + jnp.where(lane == 1, tcol, 0.0)


def _tc1(P, cnt, x, W1_l, b1, W1_r):
    return pl.pallas_call(
        _tc1_body,
        out_shape=jax.ShapeDtypeStruct((NPAD, D), jnp.float32),
    )(P, cnt, x, W1_l, b1, W1_r)


def _tc2(Q, cnt, h, W2_l, W2_r, We):
    return pl.pallas_call(
        _tc2_body,
        out_shape=jax.ShapeDtypeStruct((NPAD, D), jnp.float32),
    )(Q, cnt, h, W2_l, W2_r, We)


def kernel(x, edge_index, edge_attr, W1_l, b1, W1_r, W2_l, b2, W2_r, We, be):
    del edge_attr
    src = edge_index[0].astype(jnp.int32)
    dst = edge_index[1].astype(jnp.int32)
    edges_t = jnp.bitwise_or(src, jnp.left_shift(dst, 14)).reshape(
        NW, NCHUNK, K)

    x_pad = jnp.pad(x, ((0, NPAD - N), (0, 0)))
    cnt = _count(dst.reshape(NW, EP))
    P = _segsum(x_pad, edges_t)
    h = _tc1(P, cnt, x_pad, W1_l, b1.reshape(1, D), W1_r)
    Q = _segsum(h, edges_t)
    st = _tc2(Q, cnt, h, W2_l, W2_r, We)
    s_arr = st[:N, 0]
    t_arr = st[:N, 1]
    be16 = jnp.pad(be.astype(jnp.float32), (0, 15))
    y = _head(s_arr, t_arr, src, dst, We.reshape(2 * D), b2, be16)
    return y.reshape(E, 1)


# speedup vs baseline: 1.2085x; 1.2085x over previous
import functools

import jax
import jax.numpy as jnp
from jax import lax
from jax.experimental import pallas as pl
from jax.experimental.pallas import tpu as pltpu
from jax.experimental.pallas import tpu_sc as plsc

N = 10000
NPAD = 10112
E = 320000
D = 128
NC = 2
NS = 16
NW = NC * NS
EP = E // NW
K = 80
NCHUNK = EP // K
RP = NPAD // NS
RCS = (80, 80, 80, 80, 80, 80, 80, 72)

_mesh = plsc.VectorSubcoreMesh(core_axis_name="c", subcore_axis_name="s")
_sc_params = pltpu.CompilerParams(needs_layout_passes=False)


@functools.partial(
    pl.kernel,
    out_type=jax.ShapeDtypeStruct((NW, NPAD), jnp.float32),
    mesh=_mesh,
    scratch_types=[
        pltpu.VMEM((EP,), jnp.int32),
        pltpu.VMEM((NPAD,), jnp.float32),
    ],
    compiler_params=_sc_params,
)
def _count(dst_hbm, cnt_hbm, dst_v, cnt_v):
    c = lax.axis_index("c")
    s = lax.axis_index("s")
    wid = c * NS + s

    def _zcnt(i, _):
        cnt_v[pl.ds(i * 16, 16)] = jnp.zeros((16,), jnp.float32)
        return 0
    lax.fori_loop(0, NPAD // 16, _zcnt, 0)

    pltpu.sync_copy(dst_hbm.at[wid], dst_v)

    ones16 = jnp.ones((16,), jnp.float32)

    def _body(i, _):
        di = dst_v[pl.ds(i * 16, 16)]
        plsc.addupdate_scatter(cnt_v, [di], ones16)
        return 0
    lax.fori_loop(0, EP // 16, _body, 0)

    pltpu.sync_copy(cnt_v, cnt_hbm.at[wid])


@functools.partial(
    pl.kernel,
    out_type=jax.ShapeDtypeStruct((NC, NPAD, D), jnp.float32),
    mesh=_mesh,
    scratch_types=[
        pltpu.VMEM((NCHUNK, K), jnp.int32),
        pltpu.VMEM((K,), jnp.int32),
        pltpu.VMEM((K,), jnp.int32),
        pltpu.VMEM((K, D), jnp.float32),
        pltpu.VMEM((K,), jnp.int32),
        pltpu.VMEM((K,), jnp.int32),
        pltpu.VMEM((K, D), jnp.float32),
        pltpu.VMEM_SHARED((NPAD, D), jnp.float32),
        pltpu.SemaphoreType.DMA,
        pltpu.SemaphoreType.DMA,
        pltpu.SemaphoreType.DMA,
        pltpu.SemaphoreType.DMA,
    ],
    compiler_params=_sc_params,
)
def _segsum(tab_hbm, edges_hbm, out_hbm,
            edges_v, src_a, dst_a, rows_a, src_b, dst_b, rows_b,
            acc_shared, sem_a, sem_b, ssem_a, ssem_b):
    c = lax.axis_index("c")
    s = lax.axis_index("s")
    wid = c * NS + s

    def _zrow(i, _):
        for j in range(D // 16):
            rows_a[i, pl.ds(j * 16, 16)] = jnp.zeros((16,), jnp.float32)
        return 0
    lax.fori_loop(0, K, _zrow, 0)
    off = 0
    for rc in RCS:
        pltpu.sync_copy(rows_a.at[pl.ds(0, rc), :],
                        acc_shared.at[pl.ds(s * RP + off, rc), :])
        off += rc

    pltpu.sync_copy(edges_hbm.at[wid], edges_v)

    plsc.subcore_barrier()

    def _prefetch(j, src_c, dst_c, rows_c, sem_c):
        for m in range(K // 16):
            p = edges_v[j, pl.ds(m * 16, 16)]
            src_c[pl.ds(m * 16, 16)] = jnp.bitwise_and(p, 16383)
            dst_c[pl.ds(m * 16, 16)] = lax.shift_right_logical(p, 14)
        pltpu.async_copy(tab_hbm.at[src_c], rows_c, sem_c)

    def _wait_gather(src_c, rows_c, sem_c):
        pltpu.make_async_copy(tab_hbm.at[src_c], rows_c, sem_c).wait()

    def _scatter(dst_c, rows_c, ssem_c):
        pltpu.async_copy(rows_c, acc_shared.at[dst_c], ssem_c, add=True)

    def _wait_scatter(dst_c, rows_c, ssem_c):
        pltpu.make_async_copy(rows_c, acc_shared.at[dst_c], ssem_c).wait()

    def _drain(src_c, dst_c, rows_c, sem_c):
        _wait_gather(src_c, rows_c, sem_c)
        pltpu.sync_copy(rows_c, acc_shared.at[dst_c], add=True)

    _prefetch(0, src_a, dst_a, rows_a, sem_a)

    def _pair(i, _):
        ja = 2 * i
        _prefetch(ja + 1, src_b, dst_b, rows_b, sem_b)
        _drain(src_a, dst_a, rows_a, sem_a)

        @pl.when(ja + 2 < NCHUNK)
        def _():
            _prefetch(ja + 2, src_a, dst_a, rows_a, sem_a)
        _drain(src_b, dst_b, rows_b, sem_b)
        return 0
    lax.fori_loop(0, NCHUNK // 2, _pair, 0)

    _drain(src_a, dst_a, rows_a, sem_a)

    plsc.subcore_barrier()

    off = 0
    bufs = (rows_a, rows_b)
    for r, rc in enumerate(RCS):
        row0 = s * RP + off
        buf = bufs[r % 2]
        pltpu.sync_copy(acc_shared.at[pl.ds(row0, rc), :],
                        buf.at[pl.ds(0, rc), :])
        pltpu.sync_copy(buf.at[pl.ds(0, rc), :],
                        out_hbm.at[c, pl.ds(row0, rc), :])
        off += rc


@functools.partial(
    pl.kernel,
    out_type=jax.ShapeDtypeStruct((E,), jnp.float32),
    mesh=_mesh,
    scratch_types=[
        pltpu.VMEM((N,), jnp.float32),
        pltpu.VMEM((N,), jnp.float32),
        pltpu.VMEM((EP,), jnp.int32),
        pltpu.VMEM((EP,), jnp.int32),
        pltpu.VMEM((EP,), jnp.float32),
        pltpu.VMEM((2 * D,), jnp.float32),
        pltpu.VMEM((D,), jnp.float32),
        pltpu.VMEM((16,), jnp.float32),
    ],
    compiler_params=_sc_params,
)
def _head(s_hbm, t_hbm, src_hbm, dst_hbm, we_hbm, b2_hbm, be_hbm, out_hbm,
          s_v, t_v, src_v, dst_v, y_v, we_v, b2_v, be_v):
    c = lax.axis_index("c")
    s = lax.axis_index("s")
    wid = c * NS + s
    base = wid * EP

    pltpu.sync_copy(s_hbm, s_v)
    pltpu.sync_copy(t_hbm, t_v)
    pltpu.sync_copy(src_hbm.at[pl.ds(base, EP)], src_v)
    pltpu.sync_copy(dst_hbm.at[pl.ds(base, EP)], dst_v)
    pltpu.sync_copy(we_hbm, we_v)
    pltpu.sync_copy(b2_hbm, b2_v)
    pltpu.sync_copy(be_hbm, be_v)

    acc = be_v[...]
    for m in range(D // 16):
        b2c = b2_v[pl.ds(m * 16, 16)]
        acc = acc + b2c * (we_v[pl.ds(m * 16, 16)]
                           + we_v[pl.ds(D + m * 16, 16)])
    cconst = jnp.sum(acc, axis=0)

    def _body(i, _):
        si = src_v[pl.ds(i * 16, 16)]
        di = dst_v[pl.ds(i * 16, 16)]
        sv = plsc.load_gather(s_v, [si])
        tv = plsc.load_gather(t_v, [di])
        logit = sv + tv + cconst
        y_v[pl.ds(i * 16, 16)] = 1.0 / (1.0 + jnp.exp(-logit))
        return 0
    lax.fori_loop(0, EP // 16, _body, 0)

    pltpu.sync_copy(y_v, out_hbm.at[pl.ds(base, EP)])


def _tc1_body(p_ref, cnt_ref, x_ref, wl_ref, b_ref, wr_ref, out_ref):
    cnt = jnp.sum(cnt_ref[...], axis=0)
    inv = (1.0 / jnp.maximum(cnt, 1.0))[:, None]
    mean = (p_ref[0] + p_ref[1]) * inv
    h = lax.dot_general(mean, wl_ref[...], (((1,), (1,)), ((), ())),
                        preferred_element_type=jnp.float32)
    h = h + b_ref[...] + lax.dot_general(x_ref[...], wr_ref[...],
                                         (((1,), (1,)), ((), ())),
                                         preferred_element_type=jnp.float32)
    out_ref[...] = jnp.maximum(h, 0.0)


def _tc2_body(q_ref, cnt_ref, h_ref, wl_ref, wr_ref, we_ref, out_ref):
    cnt = jnp.sum(cnt_ref[...], axis=0)
    inv = (1.0 / jnp.maximum(cnt, 1.0))[:, None]
    mean2 = (q_ref[0] + q_ref[1]) * inv
    h = h_ref[...]
    wel = we_ref[:, :D]
    wer = we_ref[:, D:]
    cdims = (((1,), (0,)), ((), ()))

    a_s = lax.dot_general(wel, wl_ref[...], cdims,
                          preferred_element_type=jnp.float32)
    b_s = lax.dot_general(wel, wr_ref[...], cdims,
                          preferred_element_type=jnp.float32)
    a_t = lax.dot_general(wer, wl_ref[...], cdims,
                          preferred_element_type=jnp.float32)
    b_t = lax.dot_general(wer, wr_ref[...], cdims,
                          preferred_element_type=jnp.float32)
    rdims = (((1,), (1,)), ((), ()))

    scol = (lax.dot_general(mean2, a_s, rdims, preferred_element_type=jnp.float32)
            + lax.dot_general(h, b_s, rdims, preferred_element_type=jnp.float32))
    tcol = (lax.dot_general(mean2, a_t, rdims, preferred_element_type=jnp.float32)
            + lax.dot_general(h, b_t, rdims, preferred_element_type=jnp.float32))
    lane = lax.broadcasted_iota(jnp.int32, out_ref.shape, 1)
    out_ref[...] = jnp.where(lane == 0, scol, 0.0) + jnp.where(lane == 1, tcol, 0.0)


def _tc1(P, cnt, x, W1_l, b1, W1_r):
    return pl.pallas_call(
        _tc1_body,
        out_shape=jax.ShapeDtypeStruct((NPAD, D), jnp.float32),
    )(P, cnt, x, W1_l, b1, W1_r)


def _tc2(Q, cnt, h, W2_l, W2_r, We):
    return pl.pallas_call(
        _tc2_body,
        out_shape=jax.ShapeDtypeStruct((NPAD, D), jnp.float32),
    )(Q, cnt, h, W2_l, W2_r, We)


def kernel(x, edge_index, edge_attr, W1_l, b1, W1_r, W2_l, b2, W2_r, We, be):
    del edge_attr
    src = edge_index[0].astype(jnp.int32)
    dst = edge_index[1].astype(jnp.int32)
    edges_t = jnp.bitwise_or(src, jnp.left_shift(dst, 14)).reshape(
        NW, NCHUNK, K)

    x_pad = jnp.pad(x, ((0, NPAD - N), (0, 0)))
    cnt = _count(dst.reshape(NW, EP))
    P = _segsum(x_pad, edges_t)
    h = _tc1(P, cnt, x_pad, W1_l, b1.reshape(1, D), W1_r)
    Q = _segsum(h, edges_t)
    st = _tc2(Q, cnt, h, W2_l, W2_r, We)
    s_arr = st[:N, 0]
    t_arr = st[:N, 1]
    be16 = jnp.pad(be.astype(jnp.float32), (0, 15))
    y = _head(s_arr, t_arr, src, dst, We.reshape(2 * D), b2, be16)
    return y.reshape(E, 1)
